# baseline (speedup 1.0000x reference)
import functools

import jax
import jax.numpy as jnp
from jax import lax
from jax.experimental import pallas as pl
from jax.experimental.pallas import tpu as pltpu
from jax.experimental.pallas import tpu_sc as plsc

IMG_H = 512
IMG_W = 512
PADDING = 4
FEATURE_DIM = 256
N_PTS = 65536
PAD_W = IMG_W + 2 * PADDING
PAD_H = IMG_H + 2 * PADDING
TABLE_ROWS = PAD_H * PAD_W

NC = 2
NS = 16
L = 16
NW = NC * NS
PW = N_PTS // NW
CHUNK = 64
NCHUNK = PW // CHUNK


def _make_sc_kernel():
    mesh = plsc.VectorSubcoreMesh(core_axis_name="c", subcore_axis_name="s")

    @functools.partial(
        pl.kernel,
        mesh=mesh,
        out_type=jax.ShapeDtypeStruct((N_PTS, FEATURE_DIM), jnp.float32),
        scratch_types=[
            pltpu.VMEM((PW,), jnp.float32),
            pltpu.VMEM((PW,), jnp.float32),
            pltpu.VMEM((CHUNK,), jnp.int32),
            pltpu.VMEM((CHUNK,), jnp.int32),
            pltpu.VMEM((CHUNK,), jnp.int32),
            pltpu.VMEM((CHUNK,), jnp.int32),
            pltpu.VMEM((CHUNK,), jnp.float32),
            pltpu.VMEM((CHUNK,), jnp.float32),
            pltpu.VMEM((CHUNK,), jnp.float32),
            pltpu.VMEM((CHUNK,), jnp.float32),
            pltpu.VMEM((CHUNK, FEATURE_DIM), jnp.float32),
            pltpu.VMEM((CHUNK, FEATURE_DIM), jnp.float32),
            pltpu.VMEM((CHUNK, FEATURE_DIM), jnp.float32),
            pltpu.VMEM((CHUNK, FEATURE_DIM), jnp.float32),
            pltpu.VMEM((CHUNK, FEATURE_DIM), jnp.float32),
            pltpu.SemaphoreType.DMA,
        ],
    )
    def fi_kernel(y_hbm, x_hbm, table_hbm, out_hbm,
                  y_v, x_v, i00, i01, i10, i11, w00, w01, w10, w11,
                  av, bv, cv, dv, ov, sem):
        wid = lax.axis_index("s") * NC + lax.axis_index("c")
        pt_base = wid * PW
        pltpu.sync_copy(y_hbm.at[pl.ds(pt_base, PW)], y_v)
        pltpu.sync_copy(x_hbm.at[pl.ds(pt_base, PW)], x_v)

        def chunk_body(ci, carry):
            off = ci * CHUNK
            for g in range(CHUNK // L):
                s = pl.ds(g * L, L)
                yr = y_v[pl.ds(off + g * L, L)]
                xr = x_v[pl.ds(off + g * L, L)]
                y = jnp.clip(yr * jnp.float32(IMG_H) + jnp.float32(PADDING),
                             jnp.float32(0.0), jnp.float32(IMG_H - 1))
                x = jnp.clip(xr * jnp.float32(IMG_W) + jnp.float32(PADDING),
                             jnp.float32(0.0), jnp.float32(IMG_W - 1))
                yi = jnp.minimum(y.astype(jnp.int32), IMG_H - 2)
                xi = jnp.minimum(x.astype(jnp.int32), IMG_W - 2)
                yd = y - yi.astype(jnp.float32)
                xd = x - xi.astype(jnp.float32)
                base = yi * PAD_W + xi
                i00[s] = base
                i01[s] = base + 1
                i10[s] = base + PAD_W
                i11[s] = base + (PAD_W + 1)
                one = jnp.float32(1.0)
                w00[s] = (one - xd) * (one - yd)
                w01[s] = xd * (one - yd)
                w10[s] = (one - xd) * yd
                w11[s] = xd * yd
            h0 = pltpu.async_copy(table_hbm.at[i00], av, sem)
            h1 = pltpu.async_copy(table_hbm.at[i01], bv, sem)
            h2 = pltpu.async_copy(table_hbm.at[i10], cv, sem)
            h3 = pltpu.async_copy(table_hbm.at[i11], dv, sem)
            h0.wait()
            h1.wait()
            h2.wait()
            h3.wait()

            def grp_body(g, c2):
                gs = pl.ds(g * L, L)
                wa_g = w00[gs]
                wb_g = w01[gs]
                wc_g = w10[gs]
                wd_g = w11[gs]
                for lane in range(L):
                    p = g * L + lane
                    wa = wa_g[lane]
                    wb = wb_g[lane]
                    wc = wc_g[lane]
                    wd = wd_g[lane]
                    for cb in range(FEATURE_DIM // L):
                        cs = pl.ds(cb * L, L)
                        o = (wa * av[p, cs] + wb * bv[p, cs]
                             + wc * cv[p, cs] + wd * dv[p, cs])
                        ov[p, cs] = o
                return c2

            lax.fori_loop(0, CHUNK // L, grp_body, 0)
            pltpu.sync_copy(ov, out_hbm.at[pl.ds(pt_base + off, CHUNK)])
            return carry

        lax.fori_loop(0, NCHUNK, chunk_body, 0)

    return fi_kernel


_FI_KERNEL = _make_sc_kernel()


def kernel(yx, feature_img):
    y = yx[:, 0]
    x = yx[:, 1]
    table = feature_img.reshape(FEATURE_DIM, TABLE_ROWS).T
    return _FI_KERNEL(y, x, table)
